# baseline (speedup 1.0000x reference)
import jax
import jax.numpy as jnp
from jax import lax
from jax.experimental import pallas as pl
from jax.experimental.pallas import tpu as pltpu
from jax.experimental.pallas import tpu_sc as plsc

VOCAB = 100000
D = 64
B = 4096
H = 50
HP = 64

NC = 2
NS = 16
L = 16
NW = NC * NS
SPW = B // NW
C = 16
NCH = SPW // C
ROWS = C * H
DB = D // L
WROWS = SPW * HP // 128
WOFF = (0, L, 2 * L, H - L)
WLANE = [(j - WOFF[min(j // L, 3)]) for j in range(H)]


def _body(idx_hbm, w_hbm, table_hbm, out_hbm,
          idx_v, rows_v, w_v, out_v, gsems, osems):
    cid = lax.axis_index("c")
    sid = lax.axis_index("s")
    wid = sid * NC + cid
    base = wid * SPW

    pltpu.sync_copy(w_hbm.at[pl.ds(wid * WROWS, WROWS)], w_v)

    def fire_gathers(c, buf):
        pltpu.sync_copy(idx_hbm.at[pl.ds(base + c * C, C)], idx_v.at[buf])
        return [
            pltpu.async_copy(
                table_hbm.at[idx_v.at[buf, s]],
                rows_v.at[buf, pl.ds(s * H, H)],
                gsems[buf],
            )
            for s in range(C)
        ]

    def compute_chunk(c, buf):
        def body(s, _):
            gs = c * C + s
            wrow = gs // 2
            wcol = (gs % 2) * HP
            wvs = [w_v[wrow, pl.ds(wcol + o, L)] for o in WOFF]
            accs = [jnp.zeros((L,), jnp.float32) for _ in range(2 * DB)]
            for j in range(H):
                w_sj = wvs[min(j // L, 3)][WLANE[j]]
                for k in range(DB):
                    v = rows_v[buf, s * H + j, pl.ds(L * k, L)]
                    a = (j % 2) * DB + k
                    accs[a] = accs[a] + w_sj * v
            for k in range(DB):
                out_v[buf, s, pl.ds(L * k, L)] = accs[k] + accs[DB + k]
            return 0

        lax.fori_loop(0, C, body, 0)

    pending = {0: fire_gathers(0, 0)}
    out_pending = {}
    for c in range(NCH):
        buf = c % 2
        if c + 1 < NCH:
            pending[c + 1] = fire_gathers(c + 1, (c + 1) % 2)
        for cp in pending.pop(c):
            cp.wait()
        if c - 2 in out_pending:
            out_pending.pop(c - 2).wait()
        compute_chunk(c, buf)
        out_pending[c] = pltpu.async_copy(
            out_v.at[buf],
            out_hbm.at[pl.ds(base + c * C, C)],
            osems[buf],
        )
    for cp in out_pending.values():
        cp.wait()


def kernel(indices, weights, table):
    idx2 = 2 * indices
    w_p = jnp.pad(weights, ((0, 0), (0, HP - H))).reshape(B * HP // 128, 128)
    table = jnp.pad(table.T, ((0, D), (0, 0))).T.reshape(2 * VOCAB, D)

    run = pl.kernel(
        _body,
        out_type=jax.ShapeDtypeStruct((B, D), jnp.float32),
        mesh=plsc.VectorSubcoreMesh(core_axis_name="c", subcore_axis_name="s"),
        compiler_params=pltpu.CompilerParams(
            needs_layout_passes=False, use_tc_tiling_on_sc=False
        ),
        scratch_types=[
            pltpu.VMEM((2, C, H), jnp.int32),
            pltpu.VMEM((2, ROWS, D), jnp.float32),
            pltpu.VMEM((WROWS, 128), jnp.float32),
            pltpu.VMEM((2, C, D), jnp.float32),
            [pltpu.SemaphoreType.DMA, pltpu.SemaphoreType.DMA],
            [pltpu.SemaphoreType.DMA, pltpu.SemaphoreType.DMA],
        ],
    )
    return run(idx2, w_p, table)


# speedup vs baseline: 1.1400x; 1.1400x over previous
import jax
import jax.numpy as jnp
from jax import lax
from jax.experimental import pallas as pl
from jax.experimental.pallas import tpu as pltpu
from jax.experimental.pallas import tpu_sc as plsc

VOCAB = 100000
D = 64
B = 4096
H = 50
HP = 64

NC = 2
NS = 16
L = 16
NW = NC * NS
SPW = B // NW
C = 16
NCH = SPW // C
ROWS = C * H
DB = D // L
WROWS = SPW * HP // 128
WOFF = (0, L, 2 * L, H - L)
WLANE = [(j - WOFF[min(j // L, 3)]) for j in range(H)]


def _body(idx_hbm, w_hbm, table_hbm, out_hbm,
          idx_v, rows_v, w_v, out_v, gsems, osems):
    cid = lax.axis_index("c")
    sid = lax.axis_index("s")
    wid = sid * NC + cid
    base = wid * SPW

    pltpu.sync_copy(w_hbm.at[pl.ds(wid * WROWS, WROWS)], w_v)

    def fire_gathers(c, buf):
        pltpu.sync_copy(idx_hbm.at[pl.ds(base + c * C, C)], idx_v.at[buf])
        return [
            pltpu.async_copy(
                table_hbm.at[idx_v.at[buf, s]],
                rows_v.at[buf, pl.ds(s * H, H)],
                gsems[buf],
            )
            for s in range(C)
        ]

    def compute_chunk(c, buf):
        def body(s, _):
            gs = c * C + s
            wrow = gs // 2
            wcol = (gs % 2) * HP
            wvs = [w_v[wrow, pl.ds(wcol + o, L)] for o in WOFF]
            accs = [jnp.zeros((L,), jnp.float32) for _ in range(2 * DB)]
            for j in range(H):
                w_sj = wvs[min(j // L, 3)][WLANE[j]]
                for k in range(DB):
                    v = rows_v[buf, s * H + j, pl.ds(L * k, L)]
                    a = (j % 2) * DB + k
                    accs[a] = accs[a] + w_sj * v
            for k in range(DB):
                out_v[buf, s, pl.ds(L * k, L)] = accs[k] + accs[DB + k]
            return 0

        lax.fori_loop(0, C, body, 0)

    pending = {0: fire_gathers(0, 0)}
    out_pending = {}
    for c in range(NCH):
        buf = c % 2
        if c + 1 < NCH:
            pending[c + 1] = fire_gathers(c + 1, (c + 1) % 2)
        for cp in pending.pop(c):
            cp.wait()
        if c - 2 in out_pending:
            out_pending.pop(c - 2).wait()
        compute_chunk(c, buf)
        out_pending[c] = pltpu.async_copy(
            out_v.at[buf],
            out_hbm.at[pl.ds(base + c * C, C)],
            osems[buf],
        )
    for cp in out_pending.values():
        cp.wait()


def kernel(indices, weights, table):
    idx2 = 2 * indices
    w_p = jnp.pad(weights, ((0, 0), (0, HP - H))).reshape(B * HP // 128, 128)
    table = jnp.pad(table, ((0, 0), (0, D))).reshape(2 * VOCAB, D)

    run = pl.kernel(
        _body,
        out_type=jax.ShapeDtypeStruct((B, D), jnp.float32),
        mesh=plsc.VectorSubcoreMesh(core_axis_name="c", subcore_axis_name="s"),
        compiler_params=pltpu.CompilerParams(
            needs_layout_passes=False, use_tc_tiling_on_sc=False
        ),
        scratch_types=[
            pltpu.VMEM((2, C, H), jnp.int32),
            pltpu.VMEM((2, ROWS, D), jnp.float32),
            pltpu.VMEM((WROWS, 128), jnp.float32),
            pltpu.VMEM((2, C, D), jnp.float32),
            [pltpu.SemaphoreType.DMA, pltpu.SemaphoreType.DMA],
            [pltpu.SemaphoreType.DMA, pltpu.SemaphoreType.DMA],
        ],
    )
    return run(idx2, w_p, table)
